# baseline (speedup 1.0000x reference)
import jax
import jax.numpy as jnp
from jax import lax
from jax.experimental import pallas as pl
from jax.experimental.pallas import tpu as pltpu
from jax.experimental.pallas import tpu_sc as plsc

N = 100000
D = 128
B = 8192
NC = 2
NS = 16
CUT = B // 2
GPW = B // (NC * NS)
CH = 128
NCH = N // CH
TAIL = N - NCH * CH
SPS = 6272
SPL = N - (NS - 1) * SPS


def _body(ne, sid, st, en, out, gidx, ids, rows, sbuf, idt, rowst,
          cnt_v, cmat, acc, cnt_sh, gsem, isem, rsem, psem):
    c = lax.axis_index("c")
    s = lax.axis_index("s")
    w = c * NS + s

    soff = pl.multiple_of(s * SPS, 8)

    @pl.when(s < NS - 1)
    def _ps_a():
        pltpu.async_copy(sid.at[pl.ds(soff, SPS)], sbuf, psem)

    @pl.when(s == NS - 1)
    def _ps_b():
        pltpu.async_copy(sid.at[pl.ds(soff, SPL)], sbuf.at[pl.ds(0, SPL)],
                         psem)
        pltpu.async_copy(sid.at[pl.ds(NCH * CH, TAIL)],
                         sbuf.at[pl.ds(SPL, TAIL)], psem)

    base = pl.multiple_of(w * GPW, GPW)
    for h in range(2):
        pltpu.async_copy(st.at[pl.ds(base + h * 128, 128)], gidx.at[h], isem)
        pltpu.async_copy(en.at[pl.ds(base + h * 128, 128)], gidx.at[2 + h],
                         isem)
    for _ in range(4):
        pltpu.make_async_copy(st.at[pl.ds(0, 128)], gidx.at[0], isem).wait()
    for h in range(2):
        pltpu.async_copy(ne.at[gidx.at[h]], rows.at[h], gsem)

    def zero_step(i, _):
        rowst[i // 8, pl.ds((i % 8) * 16, 16)] = jnp.zeros((16,), jnp.float32)
        return 0
    lax.fori_loop(0, TAIL * 8, zero_step, 0)
    zbase = pl.multiple_of(s * 512, 512)

    def zero_copy(k, _):
        pltpu.sync_copy(rowst,
                        acc.at[pl.ds(pl.multiple_of(zbase + k * TAIL, TAIL),
                                     TAIL)])
        return 0
    lax.fori_loop(0, 512 // TAIL, zero_copy, 0)

    def count_groups(ngroups, cvec):
        def ld(i, cv):
            v = sbuf[pl.ds(i * 16, 16)]
            return cv + jnp.where(v < CUT, 1, 0).astype(jnp.int32)
        return lax.fori_loop(0, ngroups, ld, cvec)

    cvec = jnp.zeros((16,), jnp.int32)

    @pl.when(s < NS - 1)
    def _ps_wa():
        pltpu.make_async_copy(sid.at[pl.ds(0, SPS)], sbuf, psem).wait()

    @pl.when(s == NS - 1)
    def _ps_wb():
        pltpu.make_async_copy(sid.at[pl.ds(0, SPL)], sbuf.at[pl.ds(0, SPL)],
                              psem).wait()
        pltpu.make_async_copy(sid.at[pl.ds(0, TAIL)],
                              sbuf.at[pl.ds(0, TAIL)], psem).wait()

    ng = jnp.where(s < NS - 1, SPS // 16, (SPL + TAIL) // 16)
    cv = count_groups(ng, cvec)
    for j in range(8):
        cnt_v[pl.ds(16 * j, 16)] = cv

    pltpu.sync_copy(cnt_v, cnt_sh.at[s])
    plsc.subcore_barrier()
    pltpu.sync_copy(cnt_sh, cmat)

    def addrow(i, tv):
        return tv + cmat[i, pl.ds(0, 16)]
    tot = lax.fori_loop(0, NS, addrow, jnp.zeros((16,), jnp.int32))
    r_star = jnp.sum(tot)

    for _ in range(2):
        pltpu.make_async_copy(ne.at[gidx.at[0]], rows.at[0], gsem).wait()
    for h in range(2):
        pltpu.async_copy(rows.at[h],
                         out.at[pl.ds(base + h * 128, 128), pl.ds(D, D)],
                         gsem)

    nc0 = jnp.minimum((r_star + CH - 1) // CH, NCH)
    k0 = jnp.minimum(r_star // CH, NCH)
    my_lo = jnp.where(c == 0, s, k0 + s)
    my_hi = jnp.where(c == 0, nc0, NCH)
    nit = jnp.maximum(my_hi - my_lo + NS - 1, 0) // NS

    for h in range(2):
        pltpu.make_async_copy(rows.at[h],
                              out.at[pl.ds(base, 128), pl.ds(D, D)],
                              gsem).wait()

    def fire(i, slot):
        start = pl.multiple_of((my_lo + i * NS) * CH, CH)
        pltpu.async_copy(sid.at[pl.ds(start, CH)], ids.at[slot, 0], isem)
        pltpu.async_copy(ne.at[pl.ds(start, CH)], rows.at[slot], rsem)

    @pl.when(nit > 0)
    def _prime():
        fire(0, 0)

    def it(g, _):
        @pl.when(g + 1 < nit)
        def _prefetch():
            fire(g + 1, lax.rem(g + 1, 2))
        pltpu.make_async_copy(sid.at[pl.ds(0, CH)], ids.at[0, 0],
                              isem).wait()
        pltpu.make_async_copy(ne.at[pl.ds(0, CH)], rows.at[0], rsem).wait()
        slot = lax.rem(g, 2)
        pltpu.sync_copy(rows.at[slot], acc.at[ids.at[slot, 0]], add=True)
        return 0

    lax.fori_loop(0, nit, it, 0)

    @pl.when(s == NS - 1)
    def _tail():
        pltpu.sync_copy(sid.at[pl.ds(NCH * CH, TAIL)], idt.at[0])
        pltpu.sync_copy(ne.at[pl.ds(NCH * CH, TAIL)], rowst)
        pltpu.sync_copy(rowst, acc.at[idt.at[0]], add=True)

    for h in range(2):
        pltpu.async_copy(ne.at[gidx.at[2 + h]], rows.at[h], gsem)
    for _ in range(2):
        pltpu.make_async_copy(ne.at[gidx.at[0]], rows.at[0], gsem).wait()
    for h in range(2):
        pltpu.sync_copy(rows.at[h],
                        out.at[pl.ds(base + h * 128, 128), pl.ds(2 * D, D)])

    plsc.subcore_barrier()

    obase = pl.multiple_of(c * CUT + s * GPW, GPW)
    pltpu.async_copy(acc.at[pl.ds(obase, 128)], rows.at[0], rsem)
    pltpu.make_async_copy(acc.at[pl.ds(obase, 128)], rows.at[0], rsem).wait()
    pltpu.async_copy(acc.at[pl.ds(obase + 128, 128)], rows.at[1], rsem)
    pltpu.async_copy(rows.at[0],
                     out.at[pl.ds(obase, 128), pl.ds(0, D)], gsem)
    pltpu.make_async_copy(acc.at[pl.ds(obase, 128)], rows.at[1], rsem).wait()
    pltpu.async_copy(rows.at[1],
                     out.at[pl.ds(obase + 128, 128), pl.ds(0, D)], gsem)
    for _ in range(2):
        pltpu.make_async_copy(rows.at[0],
                              out.at[pl.ds(obase, 128), pl.ds(0, D)],
                              gsem).wait()


_sc_call = pl.kernel(
    _body,
    out_type=jax.ShapeDtypeStruct((B, 3 * D), jnp.float32),
    mesh=plsc.VectorSubcoreMesh(core_axis_name="c", subcore_axis_name="s"),
    scratch_types=[
        pltpu.VMEM((4, 128), jnp.int32),
        pltpu.VMEM((2, 1, CH), jnp.int32),
        pltpu.VMEM((2, CH, D), jnp.float32),
        pltpu.VMEM((SPS,), jnp.int32),
        pltpu.VMEM((1, TAIL), jnp.int32),
        pltpu.VMEM((TAIL, D), jnp.float32),
        pltpu.VMEM((128,), jnp.int32),
        pltpu.VMEM((NS, 128), jnp.int32),
        pltpu.VMEM_SHARED((B, D), jnp.float32),
        pltpu.VMEM_SHARED((NS, 128), jnp.int32),
        pltpu.SemaphoreType.DMA,
        pltpu.SemaphoreType.DMA,
        pltpu.SemaphoreType.DMA,
        pltpu.SemaphoreType.DMA,
    ],
    compiler_params=pltpu.CompilerParams(use_tc_tiling_on_sc=True,
                                         needs_layout_passes=False),
)


def kernel(node_embed, segment_ids, start_idx, end_idx):
    return _sc_call(node_embed,
                    segment_ids.astype(jnp.int32),
                    start_idx.astype(jnp.int32),
                    end_idx.astype(jnp.int32))


# speedup vs baseline: 1.0190x; 1.0190x over previous
import jax
import jax.numpy as jnp
from jax import lax
from jax.experimental import pallas as pl
from jax.experimental.pallas import tpu as pltpu
from jax.experimental.pallas import tpu_sc as plsc

N = 100000
D = 128
B = 8192
NC = 2
NS = 16
CUT = B // 2
GPW = B // (NC * NS)
CH = 128
NCH = N // CH
TAIL = N - NCH * CH
SPS = 6272
SPL = N - (NS - 1) * SPS


def _body(ne, sid, st, en, out, gidx, ids, rows, grow1, sbuf, idt, rowst,
          cnt_v, cmat, acc, cnt_sh, gsem, isem, rsem, psem, esem):
    c = lax.axis_index("c")
    s = lax.axis_index("s")
    w = c * NS + s

    soff = pl.multiple_of(s * SPS, 8)

    @pl.when(s < NS - 1)
    def _ps_a():
        pltpu.async_copy(sid.at[pl.ds(soff, SPS)], sbuf, psem)

    @pl.when(s == NS - 1)
    def _ps_b():
        pltpu.async_copy(sid.at[pl.ds(soff, SPL)], sbuf.at[pl.ds(0, SPL)],
                         psem)
        pltpu.async_copy(sid.at[pl.ds(NCH * CH, TAIL)],
                         sbuf.at[pl.ds(SPL, TAIL)], psem)

    base = pl.multiple_of(w * GPW, GPW)
    for h in range(2):
        pltpu.async_copy(st.at[pl.ds(base + h * 128, 128)], gidx.at[h], isem)
        pltpu.async_copy(en.at[pl.ds(base + h * 128, 128)], gidx.at[2 + h],
                         isem)
    for _ in range(4):
        pltpu.make_async_copy(st.at[pl.ds(0, 128)], gidx.at[0], isem).wait()
    for h in range(2):
        pltpu.async_copy(ne.at[gidx.at[h]], rows.at[h], gsem)
    pltpu.async_copy(ne.at[gidx.at[2]], grow1, esem)

    def zero_step(i, _):
        rowst[i // 8, pl.ds((i % 8) * 16, 16)] = jnp.zeros((16,), jnp.float32)
        return 0
    lax.fori_loop(0, TAIL * 8, zero_step, 0)
    zbase = pl.multiple_of(s * 512, 512)

    def zero_copy(k, _):
        pltpu.sync_copy(rowst,
                        acc.at[pl.ds(pl.multiple_of(zbase + k * TAIL, TAIL),
                                     TAIL)])
        return 0
    lax.fori_loop(0, 512 // TAIL, zero_copy, 0)

    def count_groups(ngroups, cvec):
        def ld(i, cv):
            v = sbuf[pl.ds(i * 16, 16)]
            return cv + jnp.where(v < CUT, 1, 0).astype(jnp.int32)
        return lax.fori_loop(0, ngroups, ld, cvec)

    cvec = jnp.zeros((16,), jnp.int32)

    @pl.when(s < NS - 1)
    def _ps_wa():
        pltpu.make_async_copy(sid.at[pl.ds(0, SPS)], sbuf, psem).wait()

    @pl.when(s == NS - 1)
    def _ps_wb():
        pltpu.make_async_copy(sid.at[pl.ds(0, SPL)], sbuf.at[pl.ds(0, SPL)],
                              psem).wait()
        pltpu.make_async_copy(sid.at[pl.ds(0, TAIL)],
                              sbuf.at[pl.ds(0, TAIL)], psem).wait()

    ng = jnp.where(s < NS - 1, SPS // 16, (SPL + TAIL) // 16)
    cv = count_groups(ng, cvec)
    for j in range(8):
        cnt_v[pl.ds(16 * j, 16)] = cv

    pltpu.sync_copy(cnt_v, cnt_sh.at[s])
    plsc.subcore_barrier()
    pltpu.sync_copy(cnt_sh, cmat)

    def addrow(i, tv):
        return tv + cmat[i, pl.ds(0, 16)]
    tot = lax.fori_loop(0, NS, addrow, jnp.zeros((16,), jnp.int32))
    r_star = jnp.sum(tot)

    for _ in range(2):
        pltpu.make_async_copy(ne.at[gidx.at[0]], rows.at[0], gsem).wait()
    for h in range(2):
        pltpu.async_copy(rows.at[h],
                         out.at[pl.ds(base + h * 128, 128), pl.ds(D, D)],
                         gsem)

    nc0 = jnp.minimum((r_star + CH - 1) // CH, NCH)
    k0 = jnp.minimum(r_star // CH, NCH)
    my_lo = jnp.where(c == 0, s, k0 + s)
    my_hi = jnp.where(c == 0, nc0, NCH)
    nit = jnp.maximum(my_hi - my_lo + NS - 1, 0) // NS

    for h in range(2):
        pltpu.make_async_copy(rows.at[h],
                              out.at[pl.ds(base, 128), pl.ds(D, D)],
                              gsem).wait()

    def fire(i, slot):
        start = pl.multiple_of((my_lo + i * NS) * CH, CH)
        pltpu.async_copy(sid.at[pl.ds(start, CH)], ids.at[slot, 0], isem)
        pltpu.async_copy(ne.at[pl.ds(start, CH)], rows.at[slot], rsem)

    @pl.when(nit > 0)
    def _prime():
        fire(0, 0)

    def it(g, _):
        @pl.when(g + 1 < nit)
        def _prefetch():
            fire(g + 1, lax.rem(g + 1, 2))
        pltpu.make_async_copy(sid.at[pl.ds(0, CH)], ids.at[0, 0],
                              isem).wait()
        pltpu.make_async_copy(ne.at[pl.ds(0, CH)], rows.at[0], rsem).wait()
        slot = lax.rem(g, 2)
        pltpu.sync_copy(rows.at[slot], acc.at[ids.at[slot, 0]], add=True)
        return 0

    lax.fori_loop(0, nit, it, 0)

    @pl.when(s == NS - 1)
    def _tail():
        pltpu.sync_copy(sid.at[pl.ds(NCH * CH, TAIL)], idt.at[0])
        pltpu.sync_copy(ne.at[pl.ds(NCH * CH, TAIL)], rowst)
        pltpu.sync_copy(rowst, acc.at[idt.at[0]], add=True)

    pltpu.async_copy(ne.at[gidx.at[3]], rows.at[1], gsem)
    pltpu.make_async_copy(ne.at[gidx.at[2]], grow1, esem).wait()
    pltpu.async_copy(grow1, out.at[pl.ds(base, 128), pl.ds(2 * D, D)], esem)
    pltpu.make_async_copy(ne.at[gidx.at[0]], rows.at[1], gsem).wait()
    pltpu.sync_copy(rows.at[1],
                    out.at[pl.ds(base + 128, 128), pl.ds(2 * D, D)])
    pltpu.make_async_copy(grow1, out.at[pl.ds(base, 128), pl.ds(2 * D, D)],
                          esem).wait()

    plsc.subcore_barrier()

    obase = pl.multiple_of(c * CUT + s * GPW, GPW)
    pltpu.async_copy(acc.at[pl.ds(obase, 128)], rows.at[0], rsem)
    pltpu.make_async_copy(acc.at[pl.ds(obase, 128)], rows.at[0], rsem).wait()
    pltpu.async_copy(acc.at[pl.ds(obase + 128, 128)], rows.at[1], rsem)
    pltpu.async_copy(rows.at[0],
                     out.at[pl.ds(obase, 128), pl.ds(0, D)], gsem)
    pltpu.make_async_copy(acc.at[pl.ds(obase, 128)], rows.at[1], rsem).wait()
    pltpu.async_copy(rows.at[1],
                     out.at[pl.ds(obase + 128, 128), pl.ds(0, D)], gsem)
    for _ in range(2):
        pltpu.make_async_copy(rows.at[0],
                              out.at[pl.ds(obase, 128), pl.ds(0, D)],
                              gsem).wait()


_sc_call = pl.kernel(
    _body,
    out_type=jax.ShapeDtypeStruct((B, 3 * D), jnp.float32),
    mesh=plsc.VectorSubcoreMesh(core_axis_name="c", subcore_axis_name="s"),
    scratch_types=[
        pltpu.VMEM((4, 128), jnp.int32),
        pltpu.VMEM((2, 1, CH), jnp.int32),
        pltpu.VMEM((2, CH, D), jnp.float32),
        pltpu.VMEM((CH, D), jnp.float32),
        pltpu.VMEM((SPS,), jnp.int32),
        pltpu.VMEM((1, TAIL), jnp.int32),
        pltpu.VMEM((TAIL, D), jnp.float32),
        pltpu.VMEM((128,), jnp.int32),
        pltpu.VMEM((NS, 128), jnp.int32),
        pltpu.VMEM_SHARED((B, D), jnp.float32),
        pltpu.VMEM_SHARED((NS, 128), jnp.int32),
        pltpu.SemaphoreType.DMA,
        pltpu.SemaphoreType.DMA,
        pltpu.SemaphoreType.DMA,
        pltpu.SemaphoreType.DMA,
        pltpu.SemaphoreType.DMA,
    ],
    compiler_params=pltpu.CompilerParams(use_tc_tiling_on_sc=True,
                                         needs_layout_passes=False),
)


def kernel(node_embed, segment_ids, start_idx, end_idx):
    return _sc_call(node_embed,
                    segment_ids.astype(jnp.int32),
                    start_idx.astype(jnp.int32),
                    end_idx.astype(jnp.int32))
